# Initial kernel scaffold; baseline (speedup 1.0000x reference)
import jax
import jax.numpy as jnp
from jax.experimental import pallas as pl


def kernel(tokens, table):
    raise NotImplementedError("write your pallas kernel here")


# speedup vs baseline: 2.1696x; 2.1696x over previous
import functools

import jax
import jax.numpy as jnp
from jax import lax
from jax.experimental import pallas as pl
from jax.experimental.pallas import tpu as pltpu
from jax.experimental.pallas import tpu_sc as plsc

BATCH = 1024
SEQ = 512
EMB = 384
OUT = 384
LANES = 16

G = 128
H = (G // 4) * 3
NCHUNK = SEQ // G


def _sc_body(tokens_hbm, table_hbm, out_hbm, idx_v, emb_v, out_v, sem):
    nc = plsc.get_sparse_core_info().num_cores
    wid = lax.axis_index("s") * nc + lax.axis_index("c")
    nw = nc * plsc.get_sparse_core_info().num_subcores
    spw = BATCH // nw

    @pl.loop(0, spw)
    def _sample(s):
        b = wid * spw + s

        @pl.loop(0, NCHUNK)
        def _chunk(c):
            pltpu.sync_copy(tokens_hbm.at[b, pl.ds(c * G, G)], idx_v)
            pltpu.async_copy(table_hbm.at[idx_v], emb_v, sem).wait()

            @pl.loop(0, G // 4)
            def _group(k):
                for j in range(EMB // LANES):
                    col = j * LANES
                    e0 = emb_v[4 * k + 0, pl.ds(col, LANES)]
                    e1 = emb_v[4 * k + 1, pl.ds(col, LANES)]
                    e2 = emb_v[4 * k + 2, pl.ds(col, LANES)]
                    e3 = emb_v[4 * k + 3, pl.ds(col, LANES)]
                    out_v[3 * k + 0, pl.ds(col, LANES)] = (e0 + e1) * 0.5
                    out_v[3 * k + 1, pl.ds(col, LANES)] = (e1 + e2) * 0.5
                    out_v[3 * k + 2, pl.ds(col, LANES)] = (e2 + e3) * 0.5

            pltpu.sync_copy(out_v, out_hbm.at[b, pl.ds(c * (H), H)])


@jax.jit
def _run(tokens, table):
    mesh = plsc.VectorSubcoreMesh(core_axis_name="c", subcore_axis_name="s")
    return pl.kernel(
        _sc_body,
        out_type=jax.ShapeDtypeStruct((BATCH, OUT, EMB), jnp.float32),
        mesh=mesh,
        scratch_types=[
            pltpu.VMEM((G,), jnp.int32),
            pltpu.VMEM((G, EMB), jnp.float32),
            pltpu.VMEM((H, EMB), jnp.float32),
            pltpu.SemaphoreType.DMA,
        ],
    )(tokens, table)


def kernel(tokens, table):
    return _run(tokens, table)


# speedup vs baseline: 3.8409x; 1.7703x over previous
import jax
import jax.numpy as jnp
from jax import lax
from jax.experimental import pallas as pl
from jax.experimental.pallas import tpu as pltpu
from jax.experimental.pallas import tpu_sc as plsc

BATCH = 1024
SEQ = 512
EMB = 384
OUT = 384
LANES = 16

G = 64
H = (G // 4) * 3
NCHUNK = SEQ // G


def _sc_body(tokens_hbm, table_hbm, out_hbm,
             idx_all, emb0, emb1, out0, out1, sg0, sg1, sw0, sw1):
    info = plsc.get_sparse_core_info()
    nw = info.num_cores * info.num_subcores
    wid = lax.axis_index("s") * info.num_cores + lax.axis_index("c")
    spw = BATCH // nw
    base = wid * spw
    niter = spw * NCHUNK
    embs, outs = [emb0, emb1], [out0, out1]
    sgs, sws = [sg0, sg1], [sw0, sw1]

    pltpu.sync_copy(tokens_hbm.at[pl.ds(base, spw)], idx_all)

    def gather_issue(ii, p):
        s = ii // NCHUNK
        c = lax.rem(ii, NCHUNK)
        pltpu.async_copy(
            table_hbm.at[idx_all.at[s, pl.ds(c * G, G)]], embs[p], sgs[p])

    gather_issue(0, 0)
    gather_issue(1, 1)

    @pl.loop(0, niter, step=2)
    def _pair(i):
        for p in range(2):
            ii = i + p
            s = ii // NCHUNK
            c = lax.rem(ii, NCHUNK)
            dst = out_hbm.at[base + s, pl.ds(c * H, H)]
            pltpu.make_async_copy(
                table_hbm.at[idx_all.at[s, pl.ds(c * G, G)]],
                embs[p], sgs[p]).wait()

            @pl.when(ii >= 2)
            def _drain():
                pltpu.make_async_copy(outs[p], dst, sws[p]).wait()

            emb_v, out_v = embs[p], outs[p]

            @pl.loop(0, G // 4)
            def _group(k):
                for j in range(EMB // LANES):
                    col = j * LANES
                    e0 = emb_v[4 * k + 0, pl.ds(col, LANES)]
                    e1 = emb_v[4 * k + 1, pl.ds(col, LANES)]
                    e2 = emb_v[4 * k + 2, pl.ds(col, LANES)]
                    e3 = emb_v[4 * k + 3, pl.ds(col, LANES)]
                    out_v[3 * k + 0, pl.ds(col, LANES)] = (e0 + e1) * 0.5
                    out_v[3 * k + 1, pl.ds(col, LANES)] = (e1 + e2) * 0.5
                    out_v[3 * k + 2, pl.ds(col, LANES)] = (e2 + e3) * 0.5

            pltpu.async_copy(out_v, dst, sws[p])

            @pl.when(ii + 2 < niter)
            def _prefetch():
                gather_issue(ii + 2, p)

    for p in range(2):
        pltpu.make_async_copy(
            outs[p], out_hbm.at[0, pl.ds(0, H)], sws[p]).wait()


@jax.jit
def _run(tokens, table):
    mesh = plsc.VectorSubcoreMesh(core_axis_name="c", subcore_axis_name="s")
    info = plsc.get_sparse_core_info()
    spw = BATCH // (info.num_cores * info.num_subcores)
    return pl.kernel(
        _sc_body,
        out_type=jax.ShapeDtypeStruct((BATCH, OUT, EMB), jnp.float32),
        mesh=mesh,
        scratch_types=[
            pltpu.VMEM((spw, SEQ), jnp.int32),
            pltpu.VMEM((G, EMB), jnp.float32),
            pltpu.VMEM((G, EMB), jnp.float32),
            pltpu.VMEM((H, EMB), jnp.float32),
            pltpu.VMEM((H, EMB), jnp.float32),
            pltpu.SemaphoreType.DMA,
            pltpu.SemaphoreType.DMA,
            pltpu.SemaphoreType.DMA,
            pltpu.SemaphoreType.DMA,
        ],
    )(tokens, table)


def kernel(tokens, table):
    return _run(tokens, table)
